# baseline (speedup 1.0000x reference)
import functools

import jax
import jax.numpy as jnp
from jax.experimental import pallas as pl
from jax.experimental.pallas import tpu as pltpu
from jax.experimental.pallas import tpu_sc as plsc

_SB = 2
_SSC = 4


def _mm_kernel(x_ref, e_ref, o_ref):
    e = e_ref[...].astype(jnp.bfloat16)
    for s in range(_SB):
        x = x_ref[s].astype(jnp.bfloat16)
        o_ref[s] = jax.lax.dot_general(
            e, x, (((0,), (0,)), ((), ())),
            preferred_element_type=jnp.float32)


def kernel(inputs, embedding):
    B, S, V = inputs.shape
    D = embedding.shape[1]
    S_tc = S - _SSC

    xt = jnp.transpose(inputs, (1, 2, 0))

    ot_tc = pl.pallas_call(
        _mm_kernel,
        grid=(S_tc // _SB,),
        in_specs=[
            pl.BlockSpec((_SB, V, B), lambda i: (i, 0, 0)),
            pl.BlockSpec((V, D), lambda i: (0, 0)),
        ],
        out_specs=pl.BlockSpec((_SB, D, B), lambda i: (i, 0, 0)),
        out_shape=jax.ShapeDtypeStruct((S_tc, D, B), jnp.float32),
    )(xt, embedding)

    mesh = plsc.VectorSubcoreMesh(core_axis_name="c", subcore_axis_name="s")

    def _make_sc(vbase, vlen, first):
        scratch = [
            pltpu.VMEM((vlen, 128), jnp.float32),
            pltpu.VMEM((vlen, D), jnp.float32),
            pltpu.VMEM((D, 128), jnp.float32),
        ]

        def _sc_mm(x_hbm, e_hbm, *rest):
            if first:
                out_hbm, xbuf, ebuf, obuf = rest
            else:
                p_hbm, out_hbm, xbuf, ebuf, obuf = rest
            wid = jax.lax.axis_index("s") * 2 + jax.lax.axis_index("c")
            si = wid // 8
            b0 = (wid % 8) * 128
            pltpu.sync_copy(e_hbm.at[pl.ds(vbase, vlen), :], ebuf)
            pltpu.sync_copy(
                x_hbm.at[S_tc + si, pl.ds(vbase, vlen), pl.ds(b0, 128)],
                xbuf)
            if not first:
                pltpu.sync_copy(p_hbm.at[si, :, pl.ds(b0, 128)], obuf)
            for pi in range(4):
                def body(v, accs):
                    erow = ebuf[v, :]
                    xv0 = xbuf[v, pl.ds(pi * 32, 16)]
                    xv1 = xbuf[v, pl.ds(pi * 32 + 16, 16)]
                    out = []
                    for d in range(D):
                        es = erow[d]
                        out.append(accs[2 * d] + es * xv0)
                        out.append(accs[2 * d + 1] + es * xv1)
                    return tuple(out)

                if first:
                    init = tuple(jnp.zeros((16,), jnp.float32)
                                 for _ in range(2 * D))
                else:
                    init = tuple(obuf[d, pl.ds(pi * 32 + h * 16, 16)]
                                 for d in range(D) for h in range(2))
                accs = jax.lax.fori_loop(0, vlen, body, init)
                for d in range(D):
                    obuf[d, pl.ds(pi * 32, 16)] = accs[2 * d]
                    obuf[d, pl.ds(pi * 32 + 16, 16)] = accs[2 * d + 1]
            pltpu.sync_copy(obuf, out_hbm.at[si, :, pl.ds(b0, 128)])

        return functools.partial(
            pl.kernel, mesh=mesh,
            out_type=jax.ShapeDtypeStruct((_SSC, D, B), jnp.float32),
            scratch_types=scratch)(_sc_mm)

    part = _make_sc(0, 336, True)(xt, embedding)
    part = _make_sc(336, 336, False)(xt, embedding, part)
    ot_sc = _make_sc(672, V - 672, False)(xt, embedding, part)
    ot = jnp.concatenate([ot_tc, ot_sc], axis=0)
    return jnp.transpose(ot, (2, 0, 1))


# speedup vs baseline: 2.3600x; 2.3600x over previous
import jax
import jax.numpy as jnp
from jax.experimental import pallas as pl

_SB = 2


def _mm_kernel(x_ref, e_ref, o_ref):
    e = e_ref[...].astype(jnp.bfloat16)
    for s in range(_SB):
        x = x_ref[s].astype(jnp.bfloat16)
        o_ref[s] = jax.lax.dot_general(
            e, x, (((0,), (0,)), ((), ())),
            preferred_element_type=jnp.float32)


def kernel(inputs, embedding):
    B, S, V = inputs.shape
    D = embedding.shape[1]

    xt = jnp.transpose(inputs, (1, 2, 0))

    ot = pl.pallas_call(
        _mm_kernel,
        grid=(S // _SB,),
        in_specs=[
            pl.BlockSpec((_SB, V, B), lambda i: (i, 0, 0)),
            pl.BlockSpec((V, D), lambda i: (0, 0)),
        ],
        out_specs=pl.BlockSpec((_SB, D, B), lambda i: (i, 0, 0)),
        out_shape=jax.ShapeDtypeStruct((S, D, B), jnp.float32),
    )(xt, embedding)
    return jnp.transpose(ot, (2, 0, 1))
